# baseline (speedup 1.0000x reference)
import numpy as np
import jax
import jax.numpy as jnp
from jax.experimental import pallas as pl
from jax.experimental.pallas import tpu as pltpu

_K = 3
_EPS = 1e-5


def _round_up(n, m):
    return ((n + m - 1) // m) * m


def _pool_mat(l_in, l_out):
    m = np.zeros((l_in, l_out), np.float32)
    for j in range(l_out):
        s = (j * l_in) // l_out
        e = -((-(j + 1) * l_in) // l_out)
        m[s:e, j] = 1.0 / (e - s)
    return m


def _make_body(cin, length, tb, n_hidden):
    L = length
    TB = tb
    CH = cin // 2

    def body(c_ref,
             w2_ref,
             w3_ref,
             x_ref,
             a_ref,
             w_ref,
             o_ref):
        x2 = x_ref[...].reshape(TB * 2 * CH, L).astype(jnp.bfloat16)
        y = jnp.dot(a_ref[...], x2, preferred_element_type=jnp.float32)
        h = (y[0:TB] + pltpu.roll(y[TB:2 * TB], L - 1, 1)
             + pltpu.roll(y[2 * TB:3 * TB], L - 2, 1))
        h = jnp.maximum(h + c_ref[0], 0.0)

        h2 = (w2_ref[0] * h + w2_ref[1] * pltpu.roll(h, L - 1, 1)
              + w2_ref[2] * pltpu.roll(h, L - 2, 1))
        h2 = jnp.maximum(h2 + c_ref[1], 0.0)
        h3 = (w3_ref[0] * h2 + w3_ref[1] * pltpu.roll(h2, L - 1, 1)
              + w3_ref[2] * pltpu.roll(h2, L - 2, 1))
        h3 = jnp.maximum(h3 + c_ref[2], 0.0)

        pw = w_ref[0:L, 0:n_hidden]
        bm1 = w_ref[L:L + 1, 0:n_hidden]
        wm2r = w_ref[L + 1:L + 2, 0:n_hidden]
        z = jnp.dot(h3, pw, preferred_element_type=jnp.float32)
        z = jnp.maximum(z + bm1, 0.0)
        o_ref[...] = (jnp.sum(z * wm2r, axis=1, keepdims=True) + c_ref[3])

    return body


def kernel(x, w1_full, b1, g1, beta1, mean1, var1,
           w2_full, b2, g2, beta2, mean2, var2,
           w3_full, b3, g3, beta3, mean3, var3,
           wm1, bm1, wm2, bm2, w1, w2, w3, c):
    B, Cin, L = x.shape
    hid_dim = wm1.shape[0]
    n_hidden = wm1.shape[1]
    out_dim = wm2.shape[1]
    L3 = L - 3 * (_K - 1)
    CH = Cin // 2

    TB = min(128, _round_up(B, 8))
    B_pad = _round_up(B, TB)

    x3d = x.astype(jnp.float32)
    if B_pad != B:
        x3d = jnp.pad(x3d, ((0, B_pad - B), (0, 0), (0, 0)))

    wt = w1.reshape(Cin, _K).T
    eye = jnp.eye(TB, dtype=jnp.float32)
    amat = (eye[None, :, :, None] * wt[:, None, None, :])
    amat = amat.reshape(_K * TB, TB * Cin).astype(jnp.bfloat16)

    pool = jnp.asarray(_pool_mat(L3, hid_dim))
    pw = pool @ wm1

    wrows = _round_up(L + 2, 8)
    pw_blk = jnp.concatenate(
        [pw, jnp.zeros((L - L3, n_hidden), jnp.float32),
         bm1, wm2.T, jnp.zeros((wrows - L - 2, n_hidden), jnp.float32)], axis=0)

    c4 = jnp.concatenate([c, bm2[0]])

    flops = 2 * B_pad * (_K * Cin * L + 2 * _K * L
                         + L * n_hidden + n_hidden * out_dim)
    bytes_accessed = 4 * (x3d.size + amat.size + pw_blk.size
                          + B_pad * out_dim + 2 * _K + 4)

    out = pl.pallas_call(
        _make_body(Cin, L, TB, n_hidden),
        out_shape=jax.ShapeDtypeStruct((B_pad, out_dim), jnp.float32),
        grid_spec=pltpu.PrefetchScalarGridSpec(
            num_scalar_prefetch=3,
            grid=(B_pad // TB,),
            in_specs=[
                pl.BlockSpec((TB, Cin, L), lambda b, *_: (b, 0, 0)),
                pl.BlockSpec((_K * TB, TB * Cin), lambda b, *_: (0, 0)),
                pl.BlockSpec((wrows, n_hidden), lambda b, *_: (0, 0)),
            ],
            out_specs=pl.BlockSpec((TB, out_dim), lambda b, *_: (b, 0)),
        ),
        compiler_params=pltpu.CompilerParams(
            dimension_semantics=("parallel",),
            vmem_limit_bytes=64 * 1024 * 1024,
        ),
        cost_estimate=pl.CostEstimate(flops=flops, transcendentals=0,
                                      bytes_accessed=bytes_accessed),
    )(c4, w2, w3, x3d, amat, pw_blk)

    return out[:B]


# speedup vs baseline: 1.0314x; 1.0314x over previous
import numpy as np
import jax
import jax.numpy as jnp
from jax.experimental import pallas as pl
from jax.experimental.pallas import tpu as pltpu

_K = 3
_EPS = 1e-5


def _round_up(n, m):
    return ((n + m - 1) // m) * m


def _pool_mat(l_in, l_out):
    m = np.zeros((l_in, l_out), np.float32)
    for j in range(l_out):
        s = (j * l_in) // l_out
        e = -((-(j + 1) * l_in) // l_out)
        m[s:e, j] = 1.0 / (e - s)
    return m


def _make_body(cin, length, tbh, n_hidden):
    L = length
    TBH = tbh

    def body(c_ref,
             w2_ref,
             w3_ref,
             xa_ref,
             xb_ref,
             a_ref,
             w_ref,
             o_ref):

        def half(x_ref):
            x2 = x_ref[...].reshape(TBH * cin, L).astype(jnp.bfloat16)
            y = jnp.dot(a_ref[...], x2, preferred_element_type=jnp.float32)
            h = (y[0:TBH] + pltpu.roll(y[TBH:2 * TBH], L - 1, 1)
                 + pltpu.roll(y[2 * TBH:3 * TBH], L - 2, 1))
            h = jnp.maximum(h + c_ref[0], 0.0)

            h2 = (w2_ref[0] * h + w2_ref[1] * pltpu.roll(h, L - 1, 1)
                  + w2_ref[2] * pltpu.roll(h, L - 2, 1))
            h2 = jnp.maximum(h2 + c_ref[1], 0.0)
            h3 = (w3_ref[0] * h2 + w3_ref[1] * pltpu.roll(h2, L - 1, 1)
                  + w3_ref[2] * pltpu.roll(h2, L - 2, 1))
            h3 = jnp.maximum(h3 + c_ref[2], 0.0)

            pw = w_ref[0:L, 0:n_hidden]
            bm1 = w_ref[L:L + 1, 0:n_hidden]
            wm2r = w_ref[L + 1:L + 2, 0:n_hidden]
            z = jnp.dot(h3, pw, preferred_element_type=jnp.float32)
            z = jnp.maximum(z + bm1, 0.0)
            return jnp.sum(z * wm2r, axis=1, keepdims=True) + c_ref[3]

        o_ref[0:TBH] = half(xa_ref)
        o_ref[TBH:2 * TBH] = half(xb_ref)

    return body


def kernel(x, w1_full, b1, g1, beta1, mean1, var1,
           w2_full, b2, g2, beta2, mean2, var2,
           w3_full, b3, g3, beta3, mean3, var3,
           wm1, bm1, wm2, bm2, w1, w2, w3, c):
    B, Cin, L = x.shape
    hid_dim = wm1.shape[0]
    n_hidden = wm1.shape[1]
    out_dim = wm2.shape[1]
    L3 = L - 3 * (_K - 1)
    CH = Cin // 2

    TB = min(128, _round_up(B, 8))
    B_pad = _round_up(B, TB)
    TBH = TB // 2

    x3d = x.astype(jnp.float32)
    if B_pad != B:
        x3d = jnp.pad(x3d, ((0, B_pad - B), (0, 0), (0, 0)))

    wt = w1.reshape(Cin, _K).T
    eye = jnp.eye(TBH, dtype=jnp.float32)
    amat = (eye[None, :, :, None] * wt[:, None, None, :])
    amat = amat.reshape(_K * TBH, TBH * Cin).astype(jnp.bfloat16)

    pool = jnp.asarray(_pool_mat(L3, hid_dim))
    pw = pool @ wm1

    wrows = _round_up(L + 2, 8)
    pw_blk = jnp.concatenate(
        [pw, jnp.zeros((L - L3, n_hidden), jnp.float32),
         bm1, wm2.T, jnp.zeros((wrows - L - 2, n_hidden), jnp.float32)], axis=0)

    c4 = jnp.concatenate([c, bm2[0]])

    flops = 2 * B_pad * (_K * Cin * L + 2 * _K * L
                         + L * n_hidden + n_hidden * out_dim)
    bytes_accessed = 4 * (x3d.size + amat.size + pw_blk.size
                          + B_pad * out_dim + 2 * _K + 4)

    out = pl.pallas_call(
        _make_body(Cin, L, TBH, n_hidden),
        out_shape=jax.ShapeDtypeStruct((B_pad, out_dim), jnp.float32),
        grid_spec=pltpu.PrefetchScalarGridSpec(
            num_scalar_prefetch=3,
            grid=(B_pad // TB,),
            in_specs=[
                pl.BlockSpec((TBH, Cin, L), lambda b, *_: (2 * b, 0, 0)),
                pl.BlockSpec((TBH, Cin, L), lambda b, *_: (2 * b + 1, 0, 0)),
                pl.BlockSpec((_K * TBH, TBH * Cin), lambda b, *_: (0, 0)),
                pl.BlockSpec((wrows, n_hidden), lambda b, *_: (0, 0)),
            ],
            out_specs=pl.BlockSpec((TB, out_dim), lambda b, *_: (b, 0)),
        ),
        compiler_params=pltpu.CompilerParams(
            dimension_semantics=("parallel",),
            vmem_limit_bytes=64 * 1024 * 1024,
        ),
        cost_estimate=pl.CostEstimate(flops=flops, transcendentals=0,
                                      bytes_accessed=bytes_accessed),
    )(c4, w2, w3, x3d, x3d, amat, pw_blk)

    return out[:B]


# speedup vs baseline: 1.1074x; 1.0737x over previous
import numpy as np
import jax
import jax.numpy as jnp
from jax import lax
from jax.experimental import pallas as pl
from jax.experimental.pallas import tpu as pltpu

_K = 3
_EPS = 1e-5


def _round_up(n, m):
    return ((n + m - 1) // m) * m


def _pool_mat(l_in, l_out):
    m = np.zeros((l_in, l_out), np.float32)
    for j in range(l_out):
        s = (j * l_in) // l_out
        e = -((-(j + 1) * l_in) // l_out)
        m[s:e, j] = 1.0 / (e - s)
    return m


def _make_body(cin, length, tbh, hid_dim, n_hidden):
    L = length
    TBH = tbh

    def body(c_ref,
             w2_ref,
             w3_ref,
             bm2_ref,
             w1_ref,
             xa_ref,
             xb_ref,
             pool_ref,
             wm1_ref,
             bm1_ref,
             wm2_ref,
             o_ref,
             a_scr,
             pw_scr):

        @pl.when(pl.program_id(0) == 0)
        def _build_consts():
            cols = lax.broadcasted_iota(jnp.int32, (TBH, TBH * cin), 1)
            rows = lax.broadcasted_iota(jnp.int32, (TBH, TBH * cin), 0)
            mask = (cols // cin == rows).astype(jnp.float32)
            cmod = cols % cin
            for k in range(_K):
                val = jnp.zeros((TBH, TBH * cin), jnp.float32)
                for ci in range(cin):
                    val = val + jnp.where(cmod == ci, w1_ref[ci * _K + k], 0.0)
                a_scr[k * TBH:(k + 1) * TBH, :] = (val * mask).astype(jnp.bfloat16)
            pw_scr[...] = jnp.dot(pool_ref[...], wm1_ref[...],
                                  preferred_element_type=jnp.float32)

        def half(x_ref):
            x2 = x_ref[...].reshape(TBH * cin, L).astype(jnp.bfloat16)
            y = jnp.dot(a_scr[...], x2, preferred_element_type=jnp.float32)
            h = (y[0:TBH] + pltpu.roll(y[TBH:2 * TBH], L - 1, 1)
                 + pltpu.roll(y[2 * TBH:3 * TBH], L - 2, 1))
            h = jnp.maximum(h + c_ref[0], 0.0)

            h2 = (w2_ref[0] * h + w2_ref[1] * pltpu.roll(h, L - 1, 1)
                  + w2_ref[2] * pltpu.roll(h, L - 2, 1))
            h2 = jnp.maximum(h2 + c_ref[1], 0.0)
            h3 = (w3_ref[0] * h2 + w3_ref[1] * pltpu.roll(h2, L - 1, 1)
                  + w3_ref[2] * pltpu.roll(h2, L - 2, 1))
            h3 = jnp.maximum(h3 + c_ref[2], 0.0)

            z = jnp.dot(h3, pw_scr[...], preferred_element_type=jnp.float32)
            z = jnp.maximum(z + bm1_ref[...], 0.0)
            return (jnp.dot(z, wm2_ref[...], preferred_element_type=jnp.float32)
                    + bm2_ref[0, 0])

        o_ref[0:TBH] = half(xa_ref)
        o_ref[TBH:2 * TBH] = half(xb_ref)

    return body


def kernel(x, w1_full, b1, g1, beta1, mean1, var1,
           w2_full, b2, g2, beta2, mean2, var2,
           w3_full, b3, g3, beta3, mean3, var3,
           wm1, bm1, wm2, bm2, w1, w2, w3, c):
    B, Cin, L = x.shape
    hid_dim = wm1.shape[0]
    n_hidden = wm1.shape[1]
    out_dim = wm2.shape[1]
    L3 = L - 3 * (_K - 1)

    TB = min(128, _round_up(B, 8))
    B_pad = _round_up(B, TB)
    TBH = TB // 2

    x3d = x.astype(jnp.float32)
    if B_pad != B:
        x3d = jnp.pad(x3d, ((0, B_pad - B), (0, 0), (0, 0)))

    pool_c = np.zeros((L, hid_dim), np.float32)
    pool_c[:L3] = _pool_mat(L3, hid_dim)
    pool_c = jnp.asarray(pool_c)

    flops = 2 * B_pad * (_K * Cin * L + 2 * _K * L
                         + L * n_hidden + n_hidden * out_dim)
    bytes_accessed = 4 * (x3d.size + pool_c.size + hid_dim * n_hidden
                          + n_hidden + n_hidden * out_dim
                          + B_pad * out_dim + Cin * _K + 2 * _K + 4)

    out = pl.pallas_call(
        _make_body(Cin, L, TBH, hid_dim, n_hidden),
        out_shape=jax.ShapeDtypeStruct((B_pad, out_dim), jnp.float32),
        grid_spec=pltpu.PrefetchScalarGridSpec(
            num_scalar_prefetch=5,
            grid=(B_pad // TB,),
            in_specs=[
                pl.BlockSpec((TBH, Cin, L), lambda b, *_: (2 * b, 0, 0)),
                pl.BlockSpec((TBH, Cin, L), lambda b, *_: (2 * b + 1, 0, 0)),
                pl.BlockSpec((L, hid_dim), lambda b, *_: (0, 0)),
                pl.BlockSpec((hid_dim, n_hidden), lambda b, *_: (0, 0)),
                pl.BlockSpec((1, n_hidden), lambda b, *_: (0, 0)),
                pl.BlockSpec((n_hidden, out_dim), lambda b, *_: (0, 0)),
            ],
            out_specs=pl.BlockSpec((TB, out_dim), lambda b, *_: (b, 0)),
            scratch_shapes=[
                pltpu.VMEM((_K * TBH, TBH * Cin), jnp.bfloat16),
                pltpu.VMEM((L, n_hidden), jnp.float32),
            ],
        ),
        compiler_params=pltpu.CompilerParams(
            dimension_semantics=("arbitrary",),
            vmem_limit_bytes=64 * 1024 * 1024,
        ),
        cost_estimate=pl.CostEstimate(flops=flops, transcendentals=0,
                                      bytes_accessed=bytes_accessed),
    )(c, w2, w3, bm2, w1, x3d, x3d, pool_c, wm1, bm1, wm2)

    return out[:B]


# speedup vs baseline: 1.2391x; 1.1190x over previous
import numpy as np
import jax
import jax.numpy as jnp
from jax import lax
from jax.experimental import pallas as pl
from jax.experimental.pallas import tpu as pltpu

_K = 3
_EPS = 1e-5


def _round_up(n, m):
    return ((n + m - 1) // m) * m


def _pool_mat(l_in, l_out):
    m = np.zeros((l_in, l_out), np.float32)
    for j in range(l_out):
        s = (j * l_in) // l_out
        e = -((-(j + 1) * l_in) // l_out)
        m[s:e, j] = 1.0 / (e - s)
    return m


def _make_body(cin, length, tbh, hid_dim, n_hidden):
    L = length
    TBH = tbh

    def body(c_ref,
             w2_ref,
             w3_ref,
             bm2_ref,
             w1_ref,
             xa_ref,
             xb_ref,
             pool_ref,
             wm1_ref,
             bm1_ref,
             wm2_ref,
             o_ref,
             a_scr,
             pw_scr):

        @pl.when(pl.program_id(0) == 0)
        def _build_consts():
            cols = lax.broadcasted_iota(jnp.int32, (TBH, TBH * cin), 1)
            rows = lax.broadcasted_iota(jnp.int32, (TBH, TBH * cin), 0)
            mask = (cols // cin == rows).astype(jnp.float32)
            cmod = cols % cin
            for k in range(_K):
                val = jnp.zeros((TBH, TBH * cin), jnp.float32)
                for ci in range(cin):
                    val = val + jnp.where(cmod == ci, w1_ref[ci * _K + k], 0.0)
                a_scr[k * TBH:(k + 1) * TBH, :] = (val * mask).astype(jnp.bfloat16)
            pw_scr[...] = jnp.dot(pool_ref[...], wm1_ref[...],
                                  preferred_element_type=jnp.float32)

        def half(x_ref):
            x2 = x_ref[...].reshape(TBH * cin, L).astype(jnp.bfloat16)
            y = jnp.dot(a_scr[...], x2, preferred_element_type=jnp.float32)
            h = (y[0:TBH] + pltpu.roll(y[TBH:2 * TBH], L - 1, 1)
                 + pltpu.roll(y[2 * TBH:3 * TBH], L - 2, 1))
            h = jnp.maximum(h + c_ref[0], 0.0)

            h2 = (w2_ref[0] * h + w2_ref[1] * pltpu.roll(h, L - 1, 1)
                  + w2_ref[2] * pltpu.roll(h, L - 2, 1))
            h2 = jnp.maximum(h2 + c_ref[1], 0.0)
            h3 = (w3_ref[0] * h2 + w3_ref[1] * pltpu.roll(h2, L - 1, 1)
                  + w3_ref[2] * pltpu.roll(h2, L - 2, 1))
            h3 = jnp.maximum(h3 + c_ref[2], 0.0)

            z = jnp.dot(h3, pw_scr[...], preferred_element_type=jnp.float32)
            z = jnp.maximum(z + bm1_ref[...], 0.0)
            return (jnp.dot(z, wm2_ref[...], preferred_element_type=jnp.float32)
                    + bm2_ref[0, 0])

        o_ref[0:TBH] = half(xa_ref)
        o_ref[TBH:2 * TBH] = half(xb_ref)

    return body


def kernel(x, w1_full, b1, g1, beta1, mean1, var1,
           w2_full, b2, g2, beta2, mean2, var2,
           w3_full, b3, g3, beta3, mean3, var3,
           wm1, bm1, wm2, bm2, w1, w2, w3, c):
    B, Cin, L = x.shape
    hid_dim = wm1.shape[0]
    n_hidden = wm1.shape[1]
    out_dim = wm2.shape[1]
    L3 = L - 3 * (_K - 1)

    TB = min(256, _round_up(B, 8))
    B_pad = _round_up(B, TB)
    TBH = TB // 2

    x3d = x.astype(jnp.float32)
    if B_pad != B:
        x3d = jnp.pad(x3d, ((0, B_pad - B), (0, 0), (0, 0)))

    pool_c = np.zeros((L, hid_dim), np.float32)
    pool_c[:L3] = _pool_mat(L3, hid_dim)
    pool_c = jnp.asarray(pool_c)

    flops = 2 * B_pad * (_K * Cin * L + 2 * _K * L
                         + L * n_hidden + n_hidden * out_dim)
    bytes_accessed = 4 * (x3d.size + pool_c.size + hid_dim * n_hidden
                          + n_hidden + n_hidden * out_dim
                          + B_pad * out_dim + Cin * _K + 2 * _K + 4)

    out = pl.pallas_call(
        _make_body(Cin, L, TBH, hid_dim, n_hidden),
        out_shape=jax.ShapeDtypeStruct((B_pad, out_dim), jnp.float32),
        grid_spec=pltpu.PrefetchScalarGridSpec(
            num_scalar_prefetch=5,
            grid=(B_pad // TB,),
            in_specs=[
                pl.BlockSpec((TBH, Cin, L), lambda b, *_: (2 * b, 0, 0)),
                pl.BlockSpec((TBH, Cin, L), lambda b, *_: (2 * b + 1, 0, 0)),
                pl.BlockSpec((L, hid_dim), lambda b, *_: (0, 0)),
                pl.BlockSpec((hid_dim, n_hidden), lambda b, *_: (0, 0)),
                pl.BlockSpec((1, n_hidden), lambda b, *_: (0, 0)),
                pl.BlockSpec((n_hidden, out_dim), lambda b, *_: (0, 0)),
            ],
            out_specs=pl.BlockSpec((TB, out_dim), lambda b, *_: (b, 0)),
            scratch_shapes=[
                pltpu.VMEM((_K * TBH, TBH * Cin), jnp.bfloat16),
                pltpu.VMEM((L, n_hidden), jnp.float32),
            ],
        ),
        compiler_params=pltpu.CompilerParams(
            dimension_semantics=("arbitrary",),
            vmem_limit_bytes=64 * 1024 * 1024,
        ),
        cost_estimate=pl.CostEstimate(flops=flops, transcendentals=0,
                                      bytes_accessed=bytes_accessed),
    )(c, w2, w3, bm2, w1, x3d, x3d, pool_c, wm1, bm1, wm2)

    return out[:B]
